# baseline (speedup 1.0000x reference)
import jax
import jax.numpy as jnp
from jax import lax
from jax.experimental import pallas as pl
from jax.experimental.pallas import tpu as pltpu

_NBUF = 8


def _ring_body(x_hbm, pos_ref, o_hbm, inb, outb, in_sems, out_sems):
    B, S, _D = o_hbm.shape

    def in_copy(i, slot):
        return pltpu.make_async_copy(x_hbm.at[i], inb.at[slot], in_sems.at[slot])

    def out_copy(i, slot):
        return pltpu.make_async_copy(outb.at[slot], o_hbm.at[i], out_sems.at[slot])

    for k in range(_NBUF):
        in_copy(k, k).start()

    def body(i, carry):
        slot = lax.rem(i, _NBUF)
        in_copy(i, slot).wait()

        @pl.when(i >= _NBUF)
        def _():
            out_copy(i - _NBUF, slot).wait()

        outb[slot] = inb[slot] + pos_ref[:S]
        out_copy(i, slot).start()

        @pl.when(i + _NBUF < B)
        def _():
            in_copy(i + _NBUF, slot).start()

        return carry

    lax.fori_loop(0, B, body, 0)

    for k in range(_NBUF):
        i = B - _NBUF + k
        out_copy(i, i % _NBUF).wait()


def kernel(x, pos_table):
    B, S, D = x.shape
    return pl.pallas_call(
        _ring_body,
        in_specs=[
            pl.BlockSpec(memory_space=pltpu.MemorySpace.HBM),
            pl.BlockSpec(memory_space=pltpu.MemorySpace.VMEM),
        ],
        out_specs=pl.BlockSpec(memory_space=pltpu.MemorySpace.HBM),
        out_shape=jax.ShapeDtypeStruct((B, S, D), x.dtype),
        scratch_shapes=[
            pltpu.VMEM((_NBUF, S, D), x.dtype),
            pltpu.VMEM((_NBUF, S, D), x.dtype),
            pltpu.SemaphoreType.DMA((_NBUF,)),
            pltpu.SemaphoreType.DMA((_NBUF,)),
        ],
    )(x, pos_table)


# speedup vs baseline: 1.0031x; 1.0031x over previous
import jax
import jax.numpy as jnp
from jax.experimental import pallas as pl
from jax.experimental.pallas import tpu as pltpu

_NBUF = 8


def _ring_body(x_hbm, pos_ref, o_hbm, inb, outb, in_sems, out_sems):
    B, S, _D = o_hbm.shape

    def in_copy(i):
        slot = i % _NBUF
        return pltpu.make_async_copy(x_hbm.at[i], inb.at[slot], in_sems.at[slot])

    def out_copy(i):
        slot = i % _NBUF
        return pltpu.make_async_copy(outb.at[slot], o_hbm.at[i], out_sems.at[slot])

    for k in range(_NBUF):
        in_copy(k).start()

    for i in range(B):
        slot = i % _NBUF
        in_copy(i).wait()
        if i >= _NBUF:
            out_copy(i - _NBUF).wait()
        outb[slot] = inb[slot] + pos_ref[:S]
        out_copy(i).start()
        if i + _NBUF < B:
            in_copy(i + _NBUF).start()

    for i in range(B - _NBUF, B):
        out_copy(i).wait()


def kernel(x, pos_table):
    B, S, D = x.shape
    return pl.pallas_call(
        _ring_body,
        in_specs=[
            pl.BlockSpec(memory_space=pltpu.MemorySpace.HBM),
            pl.BlockSpec(memory_space=pltpu.MemorySpace.VMEM),
        ],
        out_specs=pl.BlockSpec(memory_space=pltpu.MemorySpace.HBM),
        out_shape=jax.ShapeDtypeStruct((B, S, D), x.dtype),
        scratch_shapes=[
            pltpu.VMEM((_NBUF, S, D), x.dtype),
            pltpu.VMEM((_NBUF, S, D), x.dtype),
            pltpu.SemaphoreType.DMA((_NBUF,)),
            pltpu.SemaphoreType.DMA((_NBUF,)),
        ],
    )(x, pos_table)
